# baseline (speedup 1.0000x reference)
import functools

import jax
import jax.numpy as jnp
from jax import lax
from jax.experimental import pallas as pl
from jax.experimental.pallas import tpu as pltpu
from jax.experimental.pallas import tpu_sc as plsc

N = 16384
DIM = 64
SEL = 2
OUT_W = DIM + SEL

NC = 2
NS = 16
L = 16
NW = NC * NS
B_PER_W = N // NW


def _make_kernel():
    mesh = plsc.VectorSubcoreMesh(core_axis_name="c", subcore_axis_name="s")

    @functools.partial(
        pl.kernel,
        mesh=mesh,
        out_type=jax.ShapeDtypeStruct((N, OUT_W), jnp.float32),
        compiler_params=pltpu.CompilerParams(use_tc_tiling_on_sc=False,
                                             needs_layout_passes=False),
        scratch_types=[
            pltpu.VMEM((B_PER_W,), jnp.int32),
            pltpu.VMEM((B_PER_W,), jnp.int32),
            pltpu.VMEM((2, SEL), jnp.float32),
            pltpu.VMEM((B_PER_W, DIM), jnp.float32),
            pltpu.VMEM((B_PER_W, SEL), jnp.float32),
            pltpu.SemaphoreType.DMA,
        ],
    )
    def k(vocab_hbm, selid_hbm, table_hbm, seltab_hbm, out_hbm,
          idx_v, sid_v, seltab_v, rows_v, sel_v, sem):
        cid = lax.axis_index("c")
        scid = lax.axis_index("s")
        wid = scid * NC + cid
        base = wid * B_PER_W

        pltpu.sync_copy(vocab_hbm.at[pl.ds(base, B_PER_W)], idx_v)
        gather = pltpu.async_copy(table_hbm.at[idx_v], rows_v, sem)

        pltpu.sync_copy(selid_hbm.at[pl.ds(base, B_PER_W)], sid_v)
        pltpu.sync_copy(seltab_hbm, seltab_v)

        lanes = lax.iota(jnp.int32, L)
        zeros = jnp.zeros((L,), jnp.int32)
        ones = jnp.ones((L,), jnp.int32)

        def sel_body(i, _):
            s_ids = sid_v[pl.ds(i * L, L)]
            c0 = plsc.load_gather(seltab_v, [s_ids, zeros])
            c1 = plsc.load_gather(seltab_v, [s_ids, ones])
            rows = lanes + i * L
            plsc.store_scatter(sel_v, [rows, zeros], c0)
            plsc.store_scatter(sel_v, [rows, ones], c1)
            return 0

        lax.fori_loop(0, B_PER_W // L, sel_body, 0)

        gather.wait()

        pltpu.sync_copy(rows_v, out_hbm.at[pl.ds(base, B_PER_W), pl.ds(0, DIM)])
        pltpu.sync_copy(sel_v, out_hbm.at[pl.ds(base, B_PER_W), pl.ds(DIM, SEL)])

    return k


@jax.jit
def kernel(vocab_ids, selector_ids, table, selector_table):
    k = _make_kernel()
    return k(vocab_ids.astype(jnp.int32), selector_ids.astype(jnp.int32),
             table, selector_table.astype(jnp.float32))
